# baseline (device time: 231365 ns/iter reference)
import jax
import jax.numpy as jnp
from jax import lax
from jax.experimental import pallas as pl
from jax.experimental.pallas import tpu as pltpu

N_DEV = 32
K_SUB = 4


def kernel(x, w_mat):
    m, k_loc = x.shape
    _, n = w_mat.shape
    cm = m // N_DEV
    half = n // 2
    subw = half // K_SUB

    def body(x_ref, w_ref, out_ref, p_ref, acc_ref, recv_ref,
             send_sem, recv_sem, credit_sem):
        my = lax.axis_index("i")
        left = lax.rem(my + N_DEV - 1, N_DEV)
        right = lax.rem(my + 1, N_DEV)
        nbr_send = (right, left)
        nbr_recv = (left, right)

        barrier = pltpu.get_barrier_semaphore()
        for nbr in (left, right):
            pl.semaphore_signal(barrier, inc=1, device_id=(nbr,),
                                device_id_type=pl.DeviceIdType.MESH)
        pl.semaphore_wait(barrier, 2)

        def col0(d, j):
            return d * half + j * subw

        def contrib(c, d, j):
            return p_ref[pl.ds(c * cm, cm), col0(d, j):col0(d, j) + subw]

        def send_chunk(d, s):
            if d == 0:
                return lax.rem(my + 2 * N_DEV - s - 1, N_DEV)
            return lax.rem(my + s + 1, N_DEV)

        chains = [(d, j) for d in range(2) for j in range(K_SUB)]

        p_ref[...] = jnp.dot(x_ref[...], w_ref[...],
                             preferred_element_type=jnp.float32)

        for d, j in chains:
            acc_ref[d, j, 0, :, :] = contrib(send_chunk(d, 0), d, j)

        rdmas = {}
        for s in range(N_DEV - 1):
            slot = s % 2
            for d, j in chains:
                if s >= 2:
                    pl.semaphore_wait(credit_sem.at[d, j], 1)
                r = pltpu.make_async_remote_copy(
                    src_ref=acc_ref.at[d, j, slot],
                    dst_ref=recv_ref.at[d, j, slot],
                    send_sem=send_sem.at[d, j, slot],
                    recv_sem=recv_sem.at[d, j, slot],
                    device_id=(nbr_send[d],),
                    device_id_type=pl.DeviceIdType.MESH,
                )
                r.start()
                rdmas[(d, j, s)] = r

            for d, j in chains:
                if s < N_DEV - 2:
                    tmp = contrib(send_chunk(d, s + 1), d, j)
                else:
                    tmp = contrib(my, d, j)
                rdmas[(d, j, s)].wait_recv()
                if s < N_DEV - 2:
                    if s >= 1:
                        rdmas[(d, j, s - 1)].wait_send()
                    acc_ref[d, j, (s + 1) % 2, :, :] = (
                        recv_ref[d, j, slot, :, :] + tmp
                    )
                    if s <= N_DEV - 4:
                        pl.semaphore_signal(
                            credit_sem.at[d, j], inc=1,
                            device_id=(nbr_recv[d],),
                            device_id_type=pl.DeviceIdType.MESH,
                        )
                else:
                    out_ref[:, col0(d, j):col0(d, j) + subw] = jnp.maximum(
                        recv_ref[d, j, slot, :, :] + tmp, 0.0
                    )

        for d, j in chains:
            rdmas[(d, j, N_DEV - 3)].wait_send()
            rdmas[(d, j, N_DEV - 2)].wait_send()

    return pl.pallas_call(
        body,
        out_shape=jax.ShapeDtypeStruct((cm, n), jnp.float32),
        in_specs=[
            pl.BlockSpec(memory_space=pltpu.VMEM),
            pl.BlockSpec(memory_space=pltpu.VMEM),
        ],
        out_specs=pl.BlockSpec(memory_space=pltpu.VMEM),
        scratch_shapes=[
            pltpu.VMEM((m, n), jnp.float32),
            pltpu.VMEM((2, K_SUB, 2, cm, subw), jnp.float32),
            pltpu.VMEM((2, K_SUB, 2, cm, subw), jnp.float32),
            pltpu.SemaphoreType.DMA((2, K_SUB, 2)),
            pltpu.SemaphoreType.DMA((2, K_SUB, 2)),
            pltpu.SemaphoreType.REGULAR((2, K_SUB)),
        ],
        compiler_params=pltpu.CompilerParams(collective_id=0),
    )(x, w_mat)


# device time: 106156 ns/iter; 2.1795x vs baseline; 2.1795x over previous
import jax
import jax.numpy as jnp
from jax import lax
from jax.experimental import pallas as pl
from jax.experimental.pallas import tpu as pltpu

N_DEV = 32
K_SUB = 4

RING = [0, 8, 16, 24, 27, 19, 11, 3, 4, 12, 20, 28, 31, 23, 15, 7,
        6, 14, 22, 30, 29, 21, 13, 5, 2, 10, 18, 26, 25, 17, 9, 1]
assert sorted(RING) == list(range(32))


def kernel(x, w_mat):
    m, k_loc = x.shape
    _, n = w_mat.shape
    cm = m // N_DEV
    half = n // 2
    subw = half // K_SUB

    def body(x_ref, w_ref, ring_ref, out_ref, p_ref, acc_ref, recv_ref,
             send_sem, recv_sem, credit_sem):
        my = lax.axis_index("i")

        ring_arr = ring_ref[...]
        iota = lax.broadcasted_iota(jnp.int32, (1, N_DEV), 1)

        def ring_at(pos):
            p = lax.rem(pos + 4 * N_DEV, N_DEV)
            return jnp.sum(jnp.where(iota == p, ring_arr, 0))

        rp = jnp.sum(jnp.where(ring_arr == my, iota, 0))
        succ = ring_at(rp + 1)
        pred = ring_at(rp - 1)
        nbr_send = (succ, pred)
        nbr_recv = (pred, succ)

        barrier = pltpu.get_barrier_semaphore()
        for nbr in (pred, succ):
            pl.semaphore_signal(barrier, inc=1, device_id=(nbr,),
                                device_id_type=pl.DeviceIdType.MESH)
        pl.semaphore_wait(barrier, 2)

        def col0(d, j):
            return d * half + j * subw

        def contrib(c, d, j):
            return p_ref[pl.ds(c * cm, cm), col0(d, j):col0(d, j) + subw]

        def send_chunk(d, s):
            if d == 0:
                return ring_at(rp - s - 1)
            return ring_at(rp + s + 1)

        chains = [(d, j) for d in range(2) for j in range(K_SUB)]

        p_ref[...] = jnp.dot(x_ref[...], w_ref[...],
                             preferred_element_type=jnp.float32)

        def start_send(d, j, s):
            slot = s % 2
            r = pltpu.make_async_remote_copy(
                src_ref=acc_ref.at[d, j, slot],
                dst_ref=recv_ref.at[d, j, slot],
                send_sem=send_sem.at[d, j, slot],
                recv_sem=recv_sem.at[d, j, slot],
                device_id=(nbr_send[d],),
                device_id_type=pl.DeviceIdType.MESH,
            )
            r.start()
            return r

        rdmas = {}
        for d, j in chains:
            acc_ref[d, j, 0, :, :] = contrib(send_chunk(d, 0), d, j)
        for d, j in chains:
            rdmas[(d, j, 0)] = start_send(d, j, 0)

        for s in range(N_DEV - 1):
            slot = s % 2
            for d, j in chains:
                rdmas[(d, j, s)].wait_recv()
                if s < N_DEV - 2:
                    if s >= 1:
                        rdmas[(d, j, s - 1)].wait_send()
                    acc_ref[d, j, (s + 1) % 2, :, :] = (
                        recv_ref[d, j, slot, :, :]
                        + contrib(send_chunk(d, s + 1), d, j)
                    )
                    if s <= N_DEV - 4:
                        pl.semaphore_signal(
                            credit_sem.at[d, j], inc=1,
                            device_id=(nbr_recv[d],),
                            device_id_type=pl.DeviceIdType.MESH,
                        )
                    if s + 1 >= 2:
                        pl.semaphore_wait(credit_sem.at[d, j], 1)
                    rdmas[(d, j, s + 1)] = start_send(d, j, s + 1)
                else:
                    out_ref[:, col0(d, j):col0(d, j) + subw] = jnp.maximum(
                        recv_ref[d, j, slot, :, :] + contrib(my, d, j), 0.0
                    )

        for d, j in chains:
            rdmas[(d, j, N_DEV - 3)].wait_send()
            rdmas[(d, j, N_DEV - 2)].wait_send()

    return pl.pallas_call(
        body,
        out_shape=jax.ShapeDtypeStruct((cm, n), jnp.float32),
        in_specs=[
            pl.BlockSpec(memory_space=pltpu.VMEM),
            pl.BlockSpec(memory_space=pltpu.VMEM),
            pl.BlockSpec(memory_space=pltpu.VMEM),
        ],
        out_specs=pl.BlockSpec(memory_space=pltpu.VMEM),
        scratch_shapes=[
            pltpu.VMEM((m, n), jnp.float32),
            pltpu.VMEM((2, K_SUB, 2, cm, subw), jnp.float32),
            pltpu.VMEM((2, K_SUB, 2, cm, subw), jnp.float32),
            pltpu.SemaphoreType.DMA((2, K_SUB, 2)),
            pltpu.SemaphoreType.DMA((2, K_SUB, 2)),
            pltpu.SemaphoreType.REGULAR((2, K_SUB)),
        ],
        compiler_params=pltpu.CompilerParams(collective_id=0),
    )(x, w_mat, jnp.asarray(RING, dtype=jnp.int32).reshape(1, N_DEV))
